# Initial kernel scaffold; baseline (speedup 1.0000x reference)
import jax
import jax.numpy as jnp
from jax.experimental import pallas as pl


def kernel(node_feats, edge_index, fc_W, fc_b, convW):
    raise NotImplementedError("write your pallas kernel here")


# speedup vs baseline: 4.0517x; 4.0517x over previous
import functools

import numpy as np
import jax
import jax.numpy as jnp
from jax import lax
from jax.experimental import pallas as pl
from jax.experimental.pallas import tpu as pltpu
from jax.experimental.pallas import tpu_sc as plsc

_N = 10000
_E = 160000
_F = 256
_H = 128
_NLAYERS = 8
_ALPHA = 0.1
_THETA = 1.0

_NC = 2
_NS = 16
_CHUNK = 128
_EPT = 79 * _CHUNK
_EPAD = _NS * _EPT
_EPT2 = 40 * _CHUNK
_EPAD2 = _NC * _NS * _EPT2
_NSH = 10112
_ZR = _NSH // _NS
_RPT = 624
_TAIL_OFF = _NS * _RPT

_mesh = plsc.VectorSubcoreMesh(
    core_axis_name="c", subcore_axis_name="s", num_cores=_NC, num_subcores=_NS)

_ZCHUNKS = ((0, 128), (128, 128), (256, 128), (384, 128), (512, 120))
_WCHUNKS = ((0, 128), (128, 128), (256, 128), (384, 128), (512, 112))


@functools.partial(
    pl.kernel,
    out_type=jax.ShapeDtypeStruct((2 * _N, _H), jnp.float32),
    mesh=_mesh,
    scratch_types=[
        pltpu.VMEM((_CHUNK,), jnp.int32),
        pltpu.VMEM((_CHUNK, _H), jnp.float32),
        pltpu.VMEM((_CHUNK, _H), jnp.float32),
        pltpu.VMEM_SHARED((_NSH, _H), jnp.float32),
    ],
)
def _sc_degree(col_hbm, out_hbm, cidx, ones, zbuf, shared):
    c = lax.axis_index("c")
    s = lax.axis_index("s")

    def _init(i, carry):
        for t in range(_H // 16):
            ones[i, pl.ds(t * 16, 16)] = jnp.ones((16,), jnp.float32)
            zbuf[i, pl.ds(t * 16, 16)] = jnp.zeros((16,), jnp.float32)
        return carry

    lax.fori_loop(0, _CHUNK, _init, 0)

    zoff = s * _ZR
    for off, sz in _ZCHUNKS:
        pltpu.sync_copy(zbuf.at[pl.ds(0, sz)], shared.at[pl.ds(zoff + off, sz)])
    plsc.subcore_barrier()

    ebase = c * (_EPAD2 // 2) + s * _EPT2

    def _chunk(j, carry):
        pltpu.sync_copy(col_hbm.at[pl.ds(ebase + j * _CHUNK, _CHUNK)], cidx)
        pltpu.sync_copy(ones, shared.at[cidx], add=True)
        return carry

    lax.fori_loop(0, _EPT2 // _CHUNK, _chunk, 0)
    plsc.subcore_barrier()

    obase = s * _RPT
    for off, sz in _WCHUNKS:
        pltpu.sync_copy(shared.at[pl.ds(obase + off, sz)],
                        zbuf.at[pl.ds(0, sz)])
        pltpu.sync_copy(zbuf.at[pl.ds(0, sz)],
                        out_hbm.at[pl.ds(c * _N + obase + off, sz)])

    @pl.when(s == _NS - 1)
    def _tail():
        pltpu.sync_copy(shared.at[pl.ds(_TAIL_OFF, 16)],
                        zbuf.at[pl.ds(0, 16)])
        pltpu.sync_copy(zbuf.at[pl.ds(0, 16)],
                        out_hbm.at[pl.ds(c * _N + _TAIL_OFF, 16)])


@functools.partial(
    pl.kernel,
    out_type=jax.ShapeDtypeStruct((2 * _N, _H), jnp.float32),
    mesh=_mesh,
    scratch_types=[
        pltpu.VMEM((_CHUNK,), jnp.int32),
        pltpu.VMEM((_CHUNK,), jnp.int32),
        pltpu.VMEM((_CHUNK, _H), jnp.float32),
        pltpu.VMEM((_CHUNK, _H), jnp.float32),
        pltpu.VMEM_SHARED((_NSH, _H), jnp.float32),
        pltpu.SemaphoreType.DMA,
    ],
)
def _sc_propagate(y_hbm, rowadj_hbm, col_hbm, out_hbm,
                  ridx, cidx, rows, zbuf, shared, sem):
    c = lax.axis_index("c")
    s = lax.axis_index("s")

    def _zb(i, carry):
        for t in range(_H // 16):
            zbuf[i, pl.ds(t * 16, 16)] = jnp.zeros((16,), jnp.float32)
        return carry

    lax.fori_loop(0, _CHUNK, _zb, 0)

    zoff = s * _ZR
    for off, sz in _ZCHUNKS:
        pltpu.sync_copy(zbuf.at[pl.ds(0, sz)], shared.at[pl.ds(zoff + off, sz)])
    plsc.subcore_barrier()

    ebase = s * _EPT
    radj_base = c * _EPAD + ebase

    def _chunk(j, carry):
        pltpu.sync_copy(rowadj_hbm.at[pl.ds(radj_base + j * _CHUNK, _CHUNK)],
                        ridx)
        pltpu.sync_copy(col_hbm.at[pl.ds(ebase + j * _CHUNK, _CHUNK)], cidx)
        pltpu.async_copy(y_hbm.at[ridx], rows, sem).wait()
        pltpu.sync_copy(rows, shared.at[cidx], add=True)
        return carry

    lax.fori_loop(0, _EPT // _CHUNK, _chunk, 0)
    plsc.subcore_barrier()

    obase = s * _RPT
    for off, sz in _WCHUNKS:
        pltpu.sync_copy(shared.at[pl.ds(obase + off, sz)],
                        zbuf.at[pl.ds(0, sz)])
        pltpu.sync_copy(zbuf.at[pl.ds(0, sz)],
                        out_hbm.at[pl.ds(c * _N + obase + off, sz)])

    @pl.when(s == _NS - 1)
    def _tail():
        pltpu.sync_copy(shared.at[pl.ds(_TAIL_OFF, 16)],
                        zbuf.at[pl.ds(0, 16)])
        pltpu.sync_copy(zbuf.at[pl.ds(0, 16)],
                        out_hbm.at[pl.ds(c * _N + _TAIL_OFF, 16)])


_BN = 2000


def _elu(x):
    return jnp.where(x > 0, x, jnp.exp(jnp.minimum(x, 0.0)) - 1.0)


def _tc0_body(nf_ref, wt_ref, b_ref, deg_ref, x0_ref, y2_ref, dinv_ref):
    x = jnp.dot(nf_ref[...], wt_ref[...],
                preferred_element_type=jnp.float32) + b_ref[...]
    x = _elu(x)
    x0_ref[...] = x
    deg = deg_ref[0][:, :1] + deg_ref[1][:, :1]
    dinv1 = jnp.where(deg > 0, lax.rsqrt(deg), 0.0)
    dinv_ref[...] = jnp.broadcast_to(dinv1, dinv_ref.shape)
    y = x * dinv1
    y2_ref[0] = y[:, :_H]
    y2_ref[1] = y[:, _H:]


def _tc0(node_feats, fc_Wt, fc_b2, degpair):
    grid = _N // _BN
    return pl.pallas_call(
        _tc0_body,
        grid=(grid,),
        in_specs=[
            pl.BlockSpec((_BN, _F), lambda i: (i, 0)),
            pl.BlockSpec((_F, _F), lambda i: (0, 0)),
            pl.BlockSpec((1, _F), lambda i: (0, 0)),
            pl.BlockSpec((2, _BN, _H), lambda i: (0, i, 0)),
        ],
        out_specs=[
            pl.BlockSpec((_BN, _F), lambda i: (i, 0)),
            pl.BlockSpec((2, _BN, _H), lambda i: (0, i, 0)),
            pl.BlockSpec((_BN, 16), lambda i: (i, 0)),
        ],
        out_shape=[
            jax.ShapeDtypeStruct((_N, _F), jnp.float32),
            jax.ShapeDtypeStruct((2, _N, _H), jnp.float32),
            jax.ShapeDtypeStruct((_N, 16), jnp.float32),
        ],
    )(node_feats, fc_Wt, fc_b2, degpair)


def _tcl_body(agg2_ref, x0_ref, dinv_ref, w_ref, beta_ref,
              y2_ref, x_ref):
    dinv = dinv_ref[...][:, :1]
    agg = jnp.concatenate([agg2_ref[0], agg2_ref[1]], axis=-1)
    h = (1.0 - _ALPHA) * (agg * dinv) + _ALPHA * x0_ref[...]
    beta = beta_ref[0, 0]
    u = h * (1.0 - beta) + jnp.dot(h, w_ref[...],
                                   preferred_element_type=jnp.float32) * beta
    x = _elu(u)
    x_ref[...] = x
    y = x * dinv
    y2_ref[0] = y[:, :_H]
    y2_ref[1] = y[:, _H:]


def _tc_layer(agg2, x0, dinv16, W, beta):
    grid = _N // _BN
    return pl.pallas_call(
        _tcl_body,
        grid=(grid,),
        in_specs=[
            pl.BlockSpec((2, _BN, _H), lambda i: (0, i, 0)),
            pl.BlockSpec((_BN, _F), lambda i: (i, 0)),
            pl.BlockSpec((_BN, 16), lambda i: (i, 0)),
            pl.BlockSpec((_F, _F), lambda i: (0, 0)),
            pl.BlockSpec((1, 1), lambda i: (0, 0)),
        ],
        out_specs=[
            pl.BlockSpec((2, _BN, _H), lambda i: (0, i, 0)),
            pl.BlockSpec((_BN, _F), lambda i: (i, 0)),
        ],
        out_shape=[
            jax.ShapeDtypeStruct((2, _N, _H), jnp.float32),
            jax.ShapeDtypeStruct((_N, _F), jnp.float32),
        ],
    )(agg2, x0, dinv16, W, beta)


def kernel(node_feats, edge_index, fc_W, fc_b, convW):
    row = edge_index[0].astype(jnp.int32)
    col = edge_index[1].astype(jnp.int32)
    padn = _EPAD - _E
    rowp = jnp.concatenate([row, jnp.zeros((padn,), jnp.int32)])
    colp = jnp.concatenate([col, jnp.full((padn,), _N, jnp.int32)])
    rowadj = jnp.concatenate([rowp, rowp + _N])
    colp2 = jnp.concatenate([col, jnp.full((_EPAD2 - _E,), _N, jnp.int32)])

    degpair = _sc_degree(colp2).reshape(2, _N, _H)
    x0, y2, dinv16 = _tc0(node_feats, fc_W.T, fc_b[None, :], degpair)

    yflat = y2.reshape(2 * _N, _H)
    x = None
    for l in range(_NLAYERS):
        beta = jnp.full((1, 1), float(np.log(_THETA / (l + 1) + 1.0)),
                        jnp.float32)
        aggflat = _sc_propagate(yflat, rowadj, colp)
        y2, x = _tc_layer(aggflat.reshape(2, _N, _H), x0, dinv16,
                          convW[l], beta)
        yflat = y2.reshape(2 * _N, _H)
    return x


# speedup vs baseline: 4.3444x; 1.0722x over previous
import functools

import numpy as np
import jax
import jax.numpy as jnp
from jax import lax
from jax.experimental import pallas as pl
from jax.experimental.pallas import tpu as pltpu
from jax.experimental.pallas import tpu_sc as plsc

_N = 10000
_E = 160000
_F = 256
_H = 128
_NLAYERS = 8
_ALPHA = 0.1
_THETA = 1.0

_NC = 2
_NS = 16
_CHUNK = 128
_NCH = 80
_EPT = _NCH * _CHUNK
_EPAD = _NS * _EPT
_NCH2 = _NCH // 2
_HCH = _NCH // 2
_NB = 2
_NSH = 10112
_ZR = _NSH // _NS
_RPT = 624
_TAIL_OFF = _NS * _RPT

_mesh = plsc.VectorSubcoreMesh(
    core_axis_name="c", subcore_axis_name="s", num_cores=_NC, num_subcores=_NS)

_ZCHUNKS = ((0, 128), (128, 128), (256, 128), (384, 128), (512, 120))
_WCHUNKS = ((0, 128), (128, 128), (256, 128), (384, 128), (512, 112))


@functools.partial(
    pl.kernel,
    out_type=jax.ShapeDtypeStruct((2 * _N, _H), jnp.float32),
    mesh=_mesh,
    scratch_types=[
        pltpu.VMEM((_NCH2, _CHUNK), jnp.int32),
        pltpu.VMEM((_CHUNK, _H), jnp.float32),
        pltpu.VMEM((_CHUNK, _H), jnp.float32),
        pltpu.VMEM_SHARED((_NSH, _H), jnp.float32),
    ],
)
def _sc_degree(col_hbm, out_hbm, cidx_all, ones, zbuf, shared):
    c = lax.axis_index("c")
    s = lax.axis_index("s")

    pltpu.sync_copy(
        col_hbm.at[pl.ds(c * (_NS * _NCH2) + s * _NCH2, _NCH2)], cidx_all)

    def _init(i, carry):
        for t in range(_H // 16):
            ones[i, pl.ds(t * 16, 16)] = jnp.ones((16,), jnp.float32)
            zbuf[i, pl.ds(t * 16, 16)] = jnp.zeros((16,), jnp.float32)
        return carry

    lax.fori_loop(0, _CHUNK, _init, 0)

    zoff = s * _ZR
    for off, sz in _ZCHUNKS:
        pltpu.sync_copy(zbuf.at[pl.ds(0, sz)], shared.at[pl.ds(zoff + off, sz)])
    plsc.subcore_barrier()

    def _chunk(j, carry):
        pltpu.sync_copy(ones, shared.at[cidx_all.at[j]], add=True)
        return carry

    lax.fori_loop(0, _NCH2, _chunk, 0)
    plsc.subcore_barrier()

    obase = s * _RPT
    for off, sz in _WCHUNKS:
        pltpu.sync_copy(shared.at[pl.ds(obase + off, sz)],
                        zbuf.at[pl.ds(0, sz)])
        pltpu.sync_copy(zbuf.at[pl.ds(0, sz)],
                        out_hbm.at[pl.ds(c * _N + obase + off, sz)])

    @pl.when(s == _NS - 1)
    def _tail():
        pltpu.sync_copy(shared.at[pl.ds(_TAIL_OFF, 16)],
                        zbuf.at[pl.ds(0, 16)])
        pltpu.sync_copy(zbuf.at[pl.ds(0, 16)],
                        out_hbm.at[pl.ds(c * _N + _TAIL_OFF, 16)])


@functools.partial(
    pl.kernel,
    out_type=jax.ShapeDtypeStruct((2 * _N, _H), jnp.float32),
    mesh=_mesh,
    scratch_types=[
        pltpu.VMEM((_HCH, _CHUNK), jnp.int32),
        pltpu.VMEM((_HCH, _CHUNK), jnp.int32),
        pltpu.VMEM((_NB, _CHUNK, _H), jnp.float32),
        pltpu.VMEM_SHARED((_NSH, _H), jnp.float32),
        [pltpu.SemaphoreType.DMA] * _NB,
    ],
)
def _sc_propagate(y_hbm, rowadj_hbm, col_hbm, out_hbm,
                  ridx_h, cidx_h, rings, shared, sems):
    c = lax.axis_index("c")
    s = lax.axis_index("s")

    def _zb(i, carry):
        for t in range(_H // 16):
            rings[0, i, pl.ds(t * 16, 16)] = jnp.zeros((16,), jnp.float32)
        return carry

    lax.fori_loop(0, _CHUNK, _zb, 0)

    zoff = s * _ZR
    for off, sz in _ZCHUNKS:
        pltpu.sync_copy(rings.at[0].at[pl.ds(0, sz)],
                        shared.at[pl.ds(zoff + off, sz)])
    plsc.subcore_barrier()

    for h in range(2):
        pltpu.sync_copy(
            rowadj_hbm.at[pl.ds((c * _NS + s) * _NCH + h * _HCH, _HCH)],
            ridx_h)
        pltpu.sync_copy(col_hbm.at[pl.ds(s * _NCH + h * _HCH, _HCH)], cidx_h)
        for b in range(_NB):
            pltpu.async_copy(y_hbm.at[ridx_h.at[b]], rings.at[b], sems[b])

        @pl.loop(0, _HCH, step=_NB)
        def _outer(j0):
            for b in range(_NB):
                j = j0 + b
                pltpu.make_async_copy(y_hbm.at[pl.ds(0, _CHUNK)],
                                      rings.at[b], sems[b]).wait()
                pltpu.sync_copy(rings.at[b], shared.at[cidx_h.at[j]],
                                add=True)

                @pl.when(j + _NB < _HCH)
                def _issue():
                    pltpu.async_copy(y_hbm.at[ridx_h.at[j + _NB]],
                                     rings.at[b], sems[b])

    plsc.subcore_barrier()

    obase = s * _RPT
    for off, sz in _WCHUNKS:
        pltpu.sync_copy(shared.at[pl.ds(obase + off, sz)],
                        rings.at[0].at[pl.ds(0, sz)])
        pltpu.sync_copy(rings.at[0].at[pl.ds(0, sz)],
                        out_hbm.at[pl.ds(c * _N + obase + off, sz)])

    @pl.when(s == _NS - 1)
    def _tail():
        pltpu.sync_copy(shared.at[pl.ds(_TAIL_OFF, 16)],
                        rings.at[1].at[pl.ds(0, 16)])
        pltpu.sync_copy(rings.at[1].at[pl.ds(0, 16)],
                        out_hbm.at[pl.ds(c * _N + _TAIL_OFF, 16)])


_BN = 2000


def _elu(x):
    return jnp.where(x > 0, x, jnp.exp(jnp.minimum(x, 0.0)) - 1.0)


def _tc0_body(nf_ref, wt_ref, b_ref, deg_ref, x0_ref, y2_ref, dinv_ref):
    x = jnp.dot(nf_ref[...], wt_ref[...],
                preferred_element_type=jnp.float32) + b_ref[...]
    x = _elu(x)
    x0_ref[...] = x
    deg = deg_ref[0][:, :1] + deg_ref[1][:, :1]
    dinv1 = jnp.where(deg > 0, lax.rsqrt(deg), 0.0)
    dinv_ref[...] = jnp.broadcast_to(dinv1, dinv_ref.shape)
    y = x * dinv1
    y2_ref[0] = y[:, :_H]
    y2_ref[1] = y[:, _H:]


def _tc0(node_feats, fc_Wt, fc_b2, degpair):
    grid = _N // _BN
    return pl.pallas_call(
        _tc0_body,
        grid=(grid,),
        in_specs=[
            pl.BlockSpec((_BN, _F), lambda i: (i, 0)),
            pl.BlockSpec((_F, _F), lambda i: (0, 0)),
            pl.BlockSpec((1, _F), lambda i: (0, 0)),
            pl.BlockSpec((2, _BN, _H), lambda i: (0, i, 0)),
        ],
        out_specs=[
            pl.BlockSpec((_BN, _F), lambda i: (i, 0)),
            pl.BlockSpec((2, _BN, _H), lambda i: (0, i, 0)),
            pl.BlockSpec((_BN, 16), lambda i: (i, 0)),
        ],
        out_shape=[
            jax.ShapeDtypeStruct((_N, _F), jnp.float32),
            jax.ShapeDtypeStruct((2, _N, _H), jnp.float32),
            jax.ShapeDtypeStruct((_N, 16), jnp.float32),
        ],
    )(node_feats, fc_Wt, fc_b2, degpair)


def _tcl_body(agg2_ref, x0_ref, dinv_ref, w_ref, beta_ref,
              y2_ref, x_ref):
    dinv = dinv_ref[...][:, :1]
    agg = jnp.concatenate([agg2_ref[0], agg2_ref[1]], axis=-1)
    h = (1.0 - _ALPHA) * (agg * dinv) + _ALPHA * x0_ref[...]
    beta = beta_ref[0, 0]
    u = h * (1.0 - beta) + jnp.dot(h, w_ref[...],
                                   preferred_element_type=jnp.float32) * beta
    x = _elu(u)
    x_ref[...] = x
    y = x * dinv
    y2_ref[0] = y[:, :_H]
    y2_ref[1] = y[:, _H:]


def _tc_layer(agg2, x0, dinv16, W, beta):
    grid = _N // _BN
    return pl.pallas_call(
        _tcl_body,
        grid=(grid,),
        in_specs=[
            pl.BlockSpec((2, _BN, _H), lambda i: (0, i, 0)),
            pl.BlockSpec((_BN, _F), lambda i: (i, 0)),
            pl.BlockSpec((_BN, 16), lambda i: (i, 0)),
            pl.BlockSpec((_F, _F), lambda i: (0, 0)),
            pl.BlockSpec((1, 1), lambda i: (0, 0)),
        ],
        out_specs=[
            pl.BlockSpec((2, _BN, _H), lambda i: (0, i, 0)),
            pl.BlockSpec((_BN, _F), lambda i: (i, 0)),
        ],
        out_shape=[
            jax.ShapeDtypeStruct((2, _N, _H), jnp.float32),
            jax.ShapeDtypeStruct((_N, _F), jnp.float32),
        ],
    )(agg2, x0, dinv16, W, beta)


def kernel(node_feats, edge_index, fc_W, fc_b, convW):
    row = edge_index[0].astype(jnp.int32)
    col = edge_index[1].astype(jnp.int32)
    padn = _EPAD - _E
    rowp = jnp.concatenate([row, jnp.zeros((padn,), jnp.int32)])
    colp = jnp.concatenate([col, jnp.full((padn,), _N, jnp.int32)])
    rowadj2d = jnp.concatenate([rowp, rowp + _N]).reshape(-1, _CHUNK)
    col2d = colp.reshape(-1, _CHUNK)

    degpair = _sc_degree(col2d).reshape(2, _N, _H)
    x0, y2, dinv16 = _tc0(node_feats, fc_W.T, fc_b[None, :], degpair)

    yflat = y2.reshape(2 * _N, _H)
    x = None
    for l in range(_NLAYERS):
        beta = jnp.full((1, 1), float(np.log(_THETA / (l + 1) + 1.0)),
                        jnp.float32)
        aggflat = _sc_propagate(yflat, rowadj2d, col2d)
        y2, x = _tc_layer(aggflat.reshape(2, _N, _H), x0, dinv16,
                          convW[l], beta)
        yflat = y2.reshape(2 * _N, _H)
    return x


# speedup vs baseline: 4.4440x; 1.0229x over previous
import functools

import numpy as np
import jax
import jax.numpy as jnp
from jax import lax
from jax.experimental import pallas as pl
from jax.experimental.pallas import tpu as pltpu
from jax.experimental.pallas import tpu_sc as plsc

_N = 10000
_E = 160000
_F = 256
_H = 128
_NLAYERS = 8
_ALPHA = 0.1
_THETA = 1.0

_NC = 2
_NS = 16
_CHUNK = 128
_NCH = 80
_EPT = _NCH * _CHUNK
_EPAD = _NS * _EPT
_NCH2 = _NCH // 2
_HCH = _NCH // 2
_NB = 2
_NSH = 10112
_ZR = _NSH // _NS
_RPT = 624
_TAIL_OFF = _NS * _RPT

_mesh = plsc.VectorSubcoreMesh(
    core_axis_name="c", subcore_axis_name="s", num_cores=_NC, num_subcores=_NS)

_ZCHUNKS = ((0, 128), (128, 128), (256, 128), (384, 128), (512, 120))
_WCHUNKS = ((0, 128), (128, 128), (256, 128), (384, 128), (512, 112))


@functools.partial(
    pl.kernel,
    out_type=jax.ShapeDtypeStruct((2 * _N, _H), jnp.float32),
    mesh=_mesh,
    scratch_types=[
        pltpu.VMEM((_NCH2, _CHUNK), jnp.int32),
        pltpu.VMEM((_CHUNK, _H), jnp.float32),
        pltpu.VMEM((_CHUNK, _H), jnp.float32),
        pltpu.VMEM_SHARED((_NSH, _H), jnp.float32),
    ],
)
def _sc_degree(col_hbm, out_hbm, cidx_all, ones, zbuf, shared):
    c = lax.axis_index("c")
    s = lax.axis_index("s")

    pltpu.sync_copy(
        col_hbm.at[pl.ds(c * (_NS * _NCH2) + s * _NCH2, _NCH2)], cidx_all)

    def _init(i, carry):
        for t in range(_H // 16):
            ones[i, pl.ds(t * 16, 16)] = jnp.ones((16,), jnp.float32)
            zbuf[i, pl.ds(t * 16, 16)] = jnp.zeros((16,), jnp.float32)
        return carry

    lax.fori_loop(0, _CHUNK, _init, 0)

    zoff = s * _ZR
    for off, sz in _ZCHUNKS:
        pltpu.sync_copy(zbuf.at[pl.ds(0, sz)], shared.at[pl.ds(zoff + off, sz)])
    plsc.subcore_barrier()

    def _chunk(j, carry):
        pltpu.sync_copy(ones, shared.at[cidx_all.at[j]], add=True)
        return carry

    lax.fori_loop(0, _NCH2, _chunk, 0)
    plsc.subcore_barrier()

    obase = s * _RPT
    for off, sz in _WCHUNKS:
        pltpu.sync_copy(shared.at[pl.ds(obase + off, sz)],
                        zbuf.at[pl.ds(0, sz)])
        pltpu.sync_copy(zbuf.at[pl.ds(0, sz)],
                        out_hbm.at[pl.ds(c * _N + obase + off, sz)])

    @pl.when(s == _NS - 1)
    def _tail():
        pltpu.sync_copy(shared.at[pl.ds(_TAIL_OFF, 16)],
                        zbuf.at[pl.ds(0, 16)])
        pltpu.sync_copy(zbuf.at[pl.ds(0, 16)],
                        out_hbm.at[pl.ds(c * _N + _TAIL_OFF, 16)])


@functools.partial(
    pl.kernel,
    out_type=jax.ShapeDtypeStruct((2 * _N, _H), jnp.float32),
    mesh=_mesh,
    scratch_types=[
        pltpu.VMEM((_HCH, _CHUNK), jnp.int32),
        pltpu.VMEM((_HCH, _CHUNK), jnp.int32),
        pltpu.VMEM((_NB, _CHUNK, _H), jnp.float32),
        pltpu.VMEM_SHARED((_NSH, _H), jnp.float32),
        [pltpu.SemaphoreType.DMA] * _NB,
    ],
)
def _sc_propagate(y_hbm, rowadj_hbm, col_hbm, out_hbm,
                  ridx_h, cidx_h, rings, shared, sems):
    c = lax.axis_index("c")
    s = lax.axis_index("s")

    def _zb(i, carry):
        for t in range(_H // 16):
            rings[0, i, pl.ds(t * 16, 16)] = jnp.zeros((16,), jnp.float32)
        return carry

    lax.fori_loop(0, _CHUNK, _zb, 0)

    zoff = s * _ZR
    for off, sz in _ZCHUNKS:
        pltpu.sync_copy(rings.at[0].at[pl.ds(0, sz)],
                        shared.at[pl.ds(zoff + off, sz)])
    plsc.subcore_barrier()

    for h in range(2):
        pltpu.sync_copy(
            rowadj_hbm.at[pl.ds((c * _NS + s) * _NCH + h * _HCH, _HCH)],
            ridx_h)
        pltpu.sync_copy(col_hbm.at[pl.ds(s * _NCH + h * _HCH, _HCH)], cidx_h)
        for b in range(_NB):
            pltpu.async_copy(y_hbm.at[ridx_h.at[b]], rings.at[b], sems[b])

        @pl.loop(0, _HCH, step=_NB)
        def _outer(j0):
            for b in range(_NB):
                j = j0 + b
                pltpu.make_async_copy(y_hbm.at[pl.ds(0, _CHUNK)],
                                      rings.at[b], sems[b]).wait()
                pltpu.sync_copy(rings.at[b], shared.at[cidx_h.at[j]],
                                add=True)

                @pl.when(j + _NB < _HCH)
                def _issue():
                    pltpu.async_copy(y_hbm.at[ridx_h.at[j + _NB]],
                                     rings.at[b], sems[b])

    plsc.subcore_barrier()

    obase = s * _RPT
    for off, sz in _WCHUNKS:
        pltpu.sync_copy(shared.at[pl.ds(obase + off, sz)],
                        rings.at[0].at[pl.ds(0, sz)])
        pltpu.sync_copy(rings.at[0].at[pl.ds(0, sz)],
                        out_hbm.at[pl.ds(c * _N + obase + off, sz)])

    @pl.when(s == _NS - 1)
    def _tail():
        pltpu.sync_copy(shared.at[pl.ds(_TAIL_OFF, 16)],
                        rings.at[1].at[pl.ds(0, 16)])
        pltpu.sync_copy(rings.at[1].at[pl.ds(0, 16)],
                        out_hbm.at[pl.ds(c * _N + _TAIL_OFF, 16)])


_BN = 2000


def _elu(x):
    return jnp.where(x > 0, x, jnp.exp(jnp.minimum(x, 0.0)) - 1.0)


def _tc0_body(nf_ref, wt_ref, b_ref, deg_ref, x0_ref, y2_ref, dinv_ref):
    x = jnp.dot(nf_ref[...], wt_ref[...],
                preferred_element_type=jnp.float32) + b_ref[...]
    x = _elu(x)
    x0_ref[...] = x
    deg = deg_ref[0][:, :1] + deg_ref[1][:, :1]
    dinv1 = jnp.where(deg > 0, lax.rsqrt(deg), 0.0)
    dinv_ref[...] = jnp.broadcast_to(dinv1, dinv_ref.shape)
    y = x * dinv1
    y2_ref[0] = y[:, :_H]
    y2_ref[1] = y[:, _H:]


def _tc0(node_feats, fc_Wt, fc_b2, degpair):
    grid = _N // _BN
    return pl.pallas_call(
        _tc0_body,
        grid=(grid,),
        in_specs=[
            pl.BlockSpec((_BN, _F), lambda i: (i, 0)),
            pl.BlockSpec((_F, _F), lambda i: (0, 0)),
            pl.BlockSpec((1, _F), lambda i: (0, 0)),
            pl.BlockSpec((2, _BN, _H), lambda i: (0, i, 0)),
        ],
        out_specs=[
            pl.BlockSpec((_BN, _F), lambda i: (i, 0)),
            pl.BlockSpec((2, _BN, _H), lambda i: (0, i, 0)),
            pl.BlockSpec((_BN, 16), lambda i: (i, 0)),
        ],
        out_shape=[
            jax.ShapeDtypeStruct((_N, _F), jnp.float32),
            jax.ShapeDtypeStruct((2, _N, _H), jnp.float32),
            jax.ShapeDtypeStruct((_N, 16), jnp.float32),
        ],
    )(node_feats, fc_Wt, fc_b2, degpair)


def _tcl_body_mid(agg2_ref, x0_ref, dinv_ref, w_ref, beta_ref, y2_ref):
    dinv = dinv_ref[...][:, :1]
    agg = jnp.concatenate([agg2_ref[0], agg2_ref[1]], axis=-1)
    h = (1.0 - _ALPHA) * (agg * dinv) + _ALPHA * x0_ref[...]
    beta = beta_ref[0, 0]
    u = h * (1.0 - beta) + jnp.dot(h, w_ref[...],
                                   preferred_element_type=jnp.float32) * beta
    y = _elu(u) * dinv
    y2_ref[0] = y[:, :_H]
    y2_ref[1] = y[:, _H:]


def _tcl_body_last(agg2_ref, x0_ref, dinv_ref, w_ref, beta_ref, x_ref):
    dinv = dinv_ref[...][:, :1]
    agg = jnp.concatenate([agg2_ref[0], agg2_ref[1]], axis=-1)
    h = (1.0 - _ALPHA) * (agg * dinv) + _ALPHA * x0_ref[...]
    beta = beta_ref[0, 0]
    u = h * (1.0 - beta) + jnp.dot(h, w_ref[...],
                                   preferred_element_type=jnp.float32) * beta
    x_ref[...] = _elu(u)


def _tc_layer(agg2, x0, dinv16, W, beta, last):
    grid = _N // _BN
    if last:
        body = _tcl_body_last
        out_spec = pl.BlockSpec((_BN, _F), lambda i: (i, 0))
        out_shape = jax.ShapeDtypeStruct((_N, _F), jnp.float32)
    else:
        body = _tcl_body_mid
        out_spec = pl.BlockSpec((2, _BN, _H), lambda i: (0, i, 0))
        out_shape = jax.ShapeDtypeStruct((2, _N, _H), jnp.float32)
    return pl.pallas_call(
        body,
        grid=(grid,),
        in_specs=[
            pl.BlockSpec((2, _BN, _H), lambda i: (0, i, 0)),
            pl.BlockSpec((_BN, _F), lambda i: (i, 0)),
            pl.BlockSpec((_BN, 16), lambda i: (i, 0)),
            pl.BlockSpec((_F, _F), lambda i: (0, 0)),
            pl.BlockSpec((1, 1), lambda i: (0, 0)),
        ],
        out_specs=[out_spec],
        out_shape=[out_shape],
    )(agg2, x0, dinv16, W, beta)[0]


def kernel(node_feats, edge_index, fc_W, fc_b, convW):
    row = edge_index[0].astype(jnp.int32)
    col = edge_index[1].astype(jnp.int32)
    padn = _EPAD - _E
    rowp = jnp.concatenate([row, jnp.zeros((padn,), jnp.int32)])
    colp = jnp.concatenate([col, jnp.full((padn,), _N, jnp.int32)])
    rowadj2d = jnp.concatenate([rowp, rowp + _N]).reshape(-1, _CHUNK)
    col2d = colp.reshape(-1, _CHUNK)

    degpair = _sc_degree(col2d).reshape(2, _N, _H)
    x0, y2, dinv16 = _tc0(node_feats, fc_W.T, fc_b[None, :], degpair)

    yflat = y2.reshape(2 * _N, _H)
    for l in range(_NLAYERS):
        beta = jnp.full((1, 1), float(np.log(_THETA / (l + 1) + 1.0)),
                        jnp.float32)
        aggflat = _sc_propagate(yflat, rowadj2d, col2d)
        last = l == _NLAYERS - 1
        out = _tc_layer(aggflat.reshape(2, _N, _H), x0, dinv16,
                        convW[l], beta, last)
        if last:
            return out
        yflat = out.reshape(2 * _N, _H)
